# Initial kernel scaffold; baseline (speedup 1.0000x reference)
import jax
import jax.numpy as jnp
from jax.experimental import pallas as pl


def kernel(input, indices, weight, bias):
    raise NotImplementedError("write your pallas kernel here")


# speedup vs baseline: 942.1082x; 942.1082x over previous
import functools

import jax
import jax.numpy as jnp
import numpy as np
from jax import lax
from jax.experimental import pallas as pl
from jax.experimental.pallas import tpu as pltpu
from jax.experimental.pallas import tpu_sc as plsc

VIEWS = 360
NDETU = 512
NX = 256
NY = 256
NPIX = NX * NY
TABLE_WORDS = VIEWS * NDETU // 2
SCALE = float(2.0 * np.pi) / (2.0 * VIEWS)

NC = 2
NS = 16
L = 16
NW = NC * NS
PIX_PER_W = NPIX // NW
GROUPS = PIX_PER_W // L
CHUNK = L * VIEWS
VSTEP = 4

_MESH = plsc.VectorSubcoreMesh(
    core_axis_name="c", subcore_axis_name="s", num_cores=NC, num_subcores=NS)


@functools.partial(
    pl.kernel,
    out_type=jax.ShapeDtypeStruct((NPIX,), jnp.float32),
    mesh=_MESH,
    compiler_params=pltpu.CompilerParams(needs_layout_passes=False),
    scratch_types=[
        pltpu.VMEM((TABLE_WORDS,), jnp.int32),
        pltpu.VMEM((CHUNK,), jnp.int32),
        pltpu.VMEM((CHUNK,), jnp.int32),
        pltpu.VMEM((CHUNK,), jnp.float32),
        pltpu.VMEM((CHUNK,), jnp.float32),
        pltpu.VMEM((PIX_PER_W,), jnp.float32),
        pltpu.VMEM((PIX_PER_W,), jnp.float32),
        pltpu.SemaphoreType.DMA,
        pltpu.SemaphoreType.DMA,
        pltpu.SemaphoreType.DMA,
        pltpu.SemaphoreType.DMA,
    ],
)
def _backproject(words_hbm, idx_hbm, w_hbm, bias_hbm, out_hbm,
                 table_v, idx0, idx1, w0, w1, bias_v, out_v,
                 si0, si1, sw0, sw1):
    wid = lax.axis_index("s") * NC + lax.axis_index("c")
    base_p = wid * PIX_PER_W
    stream_base = base_p * VIEWS

    idx_bufs = (idx0, idx1)
    w_bufs = (w0, w1)
    isems = (si0, si1)
    wsems = (sw0, sw1)

    for b in range(2):
        off = stream_base + b * CHUNK
        pltpu.async_copy(idx_hbm.at[pl.ds(off, CHUNK)], idx_bufs[b], isems[b])
        pltpu.async_copy(w_hbm.at[pl.ds(off, CHUNK)], w_bufs[b], wsems[b])
    pltpu.sync_copy(words_hbm, table_v)
    pltpu.sync_copy(bias_hbm.at[pl.ds(base_p, PIX_PER_W)], bias_v)

    lane_base = lax.iota(jnp.int32, L) * VIEWS

    @pl.loop(0, GROUPS, step=2)
    def _group_pair(g):
        for b in range(2):
            gb = g + b
            ib, wb = idx_bufs[b], w_bufs[b]
            pltpu.make_async_copy(
                idx_hbm.at[pl.ds(0, CHUNK)], ib, isems[b]).wait()
            pltpu.make_async_copy(
                w_hbm.at[pl.ds(0, CHUNK)], wb, wsems[b]).wait()

            def _views(t, accs, ib=ib, wb=wb):
                lo_t = lane_base + t * VSTEP
                new = []
                for j in range(VSTEP):
                    lo = lo_t + j
                    iv = plsc.load_gather(ib, [lo])
                    wv = plsc.load_gather(wb, [lo])
                    word = plsc.load_gather(table_v, [iv >> 1])
                    odd = (iv & 1) == 1
                    bits = jnp.where(odd, word & jnp.int32(-65536), word << 16)
                    x = plsc.bitcast(bits, jnp.float32)
                    new.append(accs[j] + wv * x)
                return tuple(new)

            z = jnp.zeros((L,), jnp.float32)
            accs = lax.fori_loop(0, VIEWS // VSTEP, _views, (z,) * VSTEP)

            nxt = gb + 2

            @pl.when(nxt < GROUPS)
            def _refill(ib=ib, wb=wb, b=b, nxt=nxt):
                off = stream_base + nxt * CHUNK
                pltpu.async_copy(idx_hbm.at[pl.ds(off, CHUNK)], ib, isems[b])
                pltpu.async_copy(w_hbm.at[pl.ds(off, CHUNK)], wb, wsems[b])

            total = (accs[0] + accs[1]) + (accs[2] + accs[3])
            res = total * SCALE + bias_v[pl.ds(gb * L, L)]
            out_v[pl.ds(PIX_PER_W - L - gb * L, L)] = jnp.flip(res, 0)

    out_base = NPIX - base_p - PIX_PER_W
    pltpu.sync_copy(out_v, out_hbm.at[pl.ds(out_base, PIX_PER_W)])


def kernel(input, indices, weight, bias):
    sino = input.reshape(-1).astype(jnp.bfloat16)
    b16 = lax.bitcast_convert_type(sino, jnp.uint16).astype(jnp.uint32)
    pair = b16.reshape(-1, 2)
    words = lax.bitcast_convert_type(pair[:, 0] | (pair[:, 1] << 16),
                                     jnp.int32)
    out = _backproject(words, indices, weight, bias)
    return out.reshape(1, 1, NX, NY)


# speedup vs baseline: 1196.1751x; 1.2697x over previous
import functools

import jax
import jax.numpy as jnp
import numpy as np
from jax import lax
from jax.experimental import pallas as pl
from jax.experimental.pallas import tpu as pltpu
from jax.experimental.pallas import tpu_sc as plsc

VIEWS = 360
NDETU = 512
NX = 256
NY = 256
NPIX = NX * NY
TABLE_WORDS = VIEWS * NDETU // 2
SCALE = float(2.0 * np.pi) / (2.0 * VIEWS)

NC = 2
NS = 16
L = 16
NW = NC * NS
PIX_PER_W = NPIX // NW
GROUPS = PIX_PER_W // L
CHUNK = L * VIEWS
VSTEP = 4

_MESH = plsc.VectorSubcoreMesh(
    core_axis_name="c", subcore_axis_name="s", num_cores=NC, num_subcores=NS)


@functools.partial(
    pl.kernel,
    out_type=jax.ShapeDtypeStruct((NPIX,), jnp.float32),
    mesh=_MESH,
    compiler_params=pltpu.CompilerParams(needs_layout_passes=False),
    scratch_types=[
        pltpu.VMEM((TABLE_WORDS,), jnp.int32),
        pltpu.VMEM((CHUNK,), jnp.int32),
        pltpu.VMEM((CHUNK,), jnp.int32),
        pltpu.VMEM((CHUNK,), jnp.float32),
        pltpu.VMEM((CHUNK,), jnp.float32),
        pltpu.VMEM((PIX_PER_W,), jnp.float32),
        pltpu.VMEM((PIX_PER_W,), jnp.float32),
        pltpu.SemaphoreType.DMA,
        pltpu.SemaphoreType.DMA,
        pltpu.SemaphoreType.DMA,
        pltpu.SemaphoreType.DMA,
    ],
)
def _backproject(words_hbm, idx_hbm, w_hbm, bias_hbm, out_hbm,
                 table_v, idx0, idx1, w0, w1, bias_v, out_v,
                 si0, si1, sw0, sw1):
    wid = lax.axis_index("s") * NC + lax.axis_index("c")
    base_p = wid * PIX_PER_W
    stream_base = base_p * VIEWS

    idx_bufs = (idx0, idx1)
    w_bufs = (w0, w1)
    isems = (si0, si1)
    wsems = (sw0, sw1)

    for b in range(2):
        off = stream_base + b * CHUNK
        pltpu.async_copy(idx_hbm.at[pl.ds(off, CHUNK)], idx_bufs[b], isems[b])
        pltpu.async_copy(w_hbm.at[pl.ds(off, CHUNK)], w_bufs[b], wsems[b])
    pltpu.sync_copy(words_hbm, table_v)
    pltpu.sync_copy(bias_hbm.at[pl.ds(base_p, PIX_PER_W)], bias_v)

    lane_base = lax.iota(jnp.int32, L) * VIEWS

    @pl.loop(0, GROUPS, step=2)
    def _group_pair(g):
        for b in range(2):
            gb = g + b
            ib, wb = idx_bufs[b], w_bufs[b]
            pltpu.make_async_copy(
                idx_hbm.at[pl.ds(0, CHUNK)], ib, isems[b]).wait()
            pltpu.make_async_copy(
                w_hbm.at[pl.ds(0, CHUNK)], wb, wsems[b]).wait()

            def _views(t, accs, ib=ib, wb=wb):
                lo_t = lane_base + t * VSTEP
                new = []
                for j in range(VSTEP):
                    lo = lo_t + j
                    iv = plsc.load_gather(ib, [lo])
                    wv = plsc.load_gather(wb, [lo])
                    word = plsc.load_gather(table_v, [iv >> 1])
                    odd = (iv & 1) == 1
                    bits = jnp.where(odd, word & jnp.int32(-65536), word << 16)
                    x = plsc.bitcast(bits, jnp.float32)
                    new.append(accs[j] + wv * x)
                return tuple(new)

            z = jnp.zeros((L,), jnp.float32)
            accs = lax.fori_loop(0, VIEWS // VSTEP, _views, (z,) * VSTEP)

            nxt = gb + 2

            @pl.when(nxt < GROUPS)
            def _refill(ib=ib, wb=wb, b=b, nxt=nxt):
                off = stream_base + nxt * CHUNK
                pltpu.async_copy(idx_hbm.at[pl.ds(off, CHUNK)], ib, isems[b])
                pltpu.async_copy(w_hbm.at[pl.ds(off, CHUNK)], wb, wsems[b])

            total = (accs[0] + accs[1]) + (accs[2] + accs[3])
            res = total * SCALE + bias_v[pl.ds(gb * L, L)]
            out_v[pl.ds(PIX_PER_W - L - gb * L, L)] = jnp.flip(res, 0)

    out_base = NPIX - base_p - PIX_PER_W
    pltpu.sync_copy(out_v, out_hbm.at[pl.ds(out_base, PIX_PER_W)])


def kernel(input, indices, weight, bias):
    bits = lax.bitcast_convert_type(input, jnp.int32)
    lsb = lax.shift_right_logical(bits, 16) & 1
    r = bits + jnp.int32(0x7FFF) + lsb
    even = lax.shift_right_logical(r[..., 0::2], 16)
    odd = r[..., 1::2] & jnp.int32(-65536)
    words = (even | odd).reshape(-1)
    out = _backproject(words, indices, weight, bias)
    return out.reshape(1, 1, NX, NY)


# speedup vs baseline: 1387.4263x; 1.1599x over previous
import functools

import jax
import jax.numpy as jnp
import numpy as np
from jax import lax
from jax.experimental import pallas as pl
from jax.experimental.pallas import tpu as pltpu
from jax.experimental.pallas import tpu_sc as plsc

VIEWS = 360
NDETU = 512
NX = 256
NY = 256
NPIX = NX * NY
TABLE_WORDS = VIEWS * NDETU // 2
SCALE = float(2.0 * np.pi) / (2.0 * VIEWS)

NC = 2
NS = 16
L = 16
NW = NC * NS
PIX_PER_W = NPIX // NW
GROUPS = PIX_PER_W // L
CHUNK = L * VIEWS
VSTEP = 4

_MESH = plsc.VectorSubcoreMesh(
    core_axis_name="c", subcore_axis_name="s", num_cores=NC, num_subcores=NS)


@functools.partial(
    pl.kernel,
    out_type=jax.ShapeDtypeStruct((NPIX,), jnp.float32),
    mesh=_MESH,
    compiler_params=pltpu.CompilerParams(needs_layout_passes=False),
    scratch_types=[
        pltpu.VMEM((TABLE_WORDS,), jnp.int32),
        pltpu.VMEM((CHUNK,), jnp.int32),
        pltpu.VMEM((CHUNK,), jnp.int32),
        pltpu.VMEM((CHUNK,), jnp.float32),
        pltpu.VMEM((CHUNK,), jnp.float32),
        pltpu.VMEM((PIX_PER_W,), jnp.float32),
        pltpu.VMEM((PIX_PER_W,), jnp.float32),
        pltpu.SemaphoreType.DMA,
        pltpu.SemaphoreType.DMA,
        pltpu.SemaphoreType.DMA,
        pltpu.SemaphoreType.DMA,
    ],
)
def _backproject(words_hbm, idx_hbm, w_hbm, bias_hbm, out_hbm,
                 table_v, idx0, idx1, w0, w1, bias_v, out_v,
                 si0, si1, sw0, sw1):
    wid = lax.axis_index("s") * NC + lax.axis_index("c")
    base_p = wid * PIX_PER_W
    stream_base = base_p * VIEWS

    idx_bufs = (idx0, idx1)
    w_bufs = (w0, w1)
    isems = (si0, si1)
    wsems = (sw0, sw1)

    for b in range(2):
        off = stream_base + b * CHUNK
        pltpu.async_copy(idx_hbm.at[pl.ds(off, CHUNK)], idx_bufs[b], isems[b])
        pltpu.async_copy(w_hbm.at[pl.ds(off, CHUNK)], w_bufs[b], wsems[b])
    pltpu.sync_copy(words_hbm, table_v)
    pltpu.sync_copy(bias_hbm.at[pl.ds(base_p, PIX_PER_W)], bias_v)

    lane_base = lax.iota(jnp.int32, L) * VIEWS

    @pl.loop(0, GROUPS, step=2)
    def _group_pair(g):
        for b in range(2):
            gb = g + b
            ib, wb = idx_bufs[b], w_bufs[b]
            pltpu.make_async_copy(
                idx_hbm.at[pl.ds(0, CHUNK)], ib, isems[b]).wait()
            pltpu.make_async_copy(
                w_hbm.at[pl.ds(0, CHUNK)], wb, wsems[b]).wait()

            def _views(t, accs, ib=ib, wb=wb):
                lo_t = lane_base + t * VSTEP
                new = []
                for j in range(VSTEP):
                    lo = lo_t + j
                    iv = plsc.load_gather(ib, [lo])
                    wv = plsc.load_gather(wb, [lo])
                    hi = iv >= TABLE_WORDS
                    word = plsc.load_gather(
                        table_v, [jnp.where(hi, iv - TABLE_WORDS, iv)])
                    bits = jnp.where(hi, word & jnp.int32(-65536), word << 16)
                    x = plsc.bitcast(bits, jnp.float32)
                    new.append(accs[j] + wv * x)
                return tuple(new)

            z = jnp.zeros((L,), jnp.float32)
            accs = lax.fori_loop(0, VIEWS // VSTEP, _views, (z,) * VSTEP)

            nxt = gb + 2

            @pl.when(nxt < GROUPS)
            def _refill(ib=ib, wb=wb, b=b, nxt=nxt):
                off = stream_base + nxt * CHUNK
                pltpu.async_copy(idx_hbm.at[pl.ds(off, CHUNK)], ib, isems[b])
                pltpu.async_copy(w_hbm.at[pl.ds(off, CHUNK)], wb, wsems[b])

            total = (accs[0] + accs[1]) + (accs[2] + accs[3])
            res = total * SCALE + bias_v[pl.ds(gb * L, L)]
            out_v[pl.ds(PIX_PER_W - L - gb * L, L)] = jnp.flip(res, 0)

    out_base = NPIX - base_p - PIX_PER_W
    pltpu.sync_copy(out_v, out_hbm.at[pl.ds(out_base, PIX_PER_W)])


def kernel(input, indices, weight, bias):
    flat = input.reshape(-1)
    bits = lax.bitcast_convert_type(flat, jnp.int32)
    lsb = lax.shift_right_logical(bits, 16) & 1
    r = bits + jnp.int32(0x7FFF) + lsb
    words = (lax.shift_right_logical(r[:TABLE_WORDS], 16)
             | (r[TABLE_WORDS:] & jnp.int32(-65536)))
    out = _backproject(words, indices, weight, bias)
    return out.reshape(1, 1, NX, NY)


# speedup vs baseline: 1402.2360x; 1.0107x over previous
import functools

import jax
import jax.numpy as jnp
import numpy as np
from jax import lax
from jax.experimental import pallas as pl
from jax.experimental.pallas import tpu as pltpu
from jax.experimental.pallas import tpu_sc as plsc

VIEWS = 360
NDETU = 512
NX = 256
NY = 256
NPIX = NX * NY
TABLE_WORDS = VIEWS * NDETU // 2
SCALE = float(2.0 * np.pi) / (2.0 * VIEWS)

NC = 2
NS = 16
L = 16
NW = NC * NS
PIX_PER_W = NPIX // NW
GROUPS = PIX_PER_W // L
CHUNK = L * VIEWS
VSTEP = 4
UNROLL = 2

_MESH = plsc.VectorSubcoreMesh(
    core_axis_name="c", subcore_axis_name="s", num_cores=NC, num_subcores=NS)


@functools.partial(
    pl.kernel,
    out_type=jax.ShapeDtypeStruct((NPIX,), jnp.float32),
    mesh=_MESH,
    compiler_params=pltpu.CompilerParams(needs_layout_passes=False),
    scratch_types=[
        pltpu.VMEM((TABLE_WORDS,), jnp.int32),
        pltpu.VMEM((CHUNK,), jnp.int32),
        pltpu.VMEM((CHUNK,), jnp.int32),
        pltpu.VMEM((CHUNK,), jnp.float32),
        pltpu.VMEM((CHUNK,), jnp.float32),
        pltpu.VMEM((PIX_PER_W,), jnp.float32),
        pltpu.VMEM((PIX_PER_W,), jnp.float32),
        pltpu.SemaphoreType.DMA,
        pltpu.SemaphoreType.DMA,
        pltpu.SemaphoreType.DMA,
        pltpu.SemaphoreType.DMA,
    ],
)
def _backproject(words_hbm, idx_hbm, w_hbm, bias_hbm, out_hbm,
                 table_v, idx0, idx1, w0, w1, bias_v, out_v,
                 si0, si1, sw0, sw1):
    wid = lax.axis_index("s") * NC + lax.axis_index("c")
    base_p = wid * PIX_PER_W
    stream_base = base_p * VIEWS

    idx_bufs = (idx0, idx1)
    w_bufs = (w0, w1)
    isems = (si0, si1)
    wsems = (sw0, sw1)

    for b in range(2):
        off = stream_base + b * CHUNK
        pltpu.async_copy(idx_hbm.at[pl.ds(off, CHUNK)], idx_bufs[b], isems[b])
        pltpu.async_copy(w_hbm.at[pl.ds(off, CHUNK)], w_bufs[b], wsems[b])
    pltpu.sync_copy(words_hbm, table_v)
    pltpu.sync_copy(bias_hbm.at[pl.ds(base_p, PIX_PER_W)], bias_v)

    lane_base = lax.iota(jnp.int32, L) * VIEWS

    @pl.loop(0, GROUPS, step=2)
    def _group_pair(g):
        for b in range(2):
            gb = g + b
            ib, wb = idx_bufs[b], w_bufs[b]
            pltpu.make_async_copy(
                idx_hbm.at[pl.ds(0, CHUNK)], ib, isems[b]).wait()
            pltpu.make_async_copy(
                w_hbm.at[pl.ds(0, CHUNK)], wb, wsems[b]).wait()

            z = jnp.zeros((L,), jnp.float32)

            @plsc.parallel_loop(0, VIEWS, step=VSTEP, unroll=UNROLL,
                                carry=(z,) * VSTEP)
            def _views(t, accs, ib=ib, wb=wb):
                lo_t = lane_base + t
                new = []
                for j in range(VSTEP):
                    lo = lo_t + j
                    iv = plsc.load_gather(ib, [lo])
                    wv = plsc.load_gather(wb, [lo])
                    hi = iv >= TABLE_WORDS
                    word = plsc.load_gather(
                        table_v, [jnp.where(hi, iv - TABLE_WORDS, iv)])
                    bits = jnp.where(hi, word & jnp.int32(-65536), word << 16)
                    x = plsc.bitcast(bits, jnp.float32)
                    new.append(accs[j] + wv * x)
                return tuple(new)

            accs = _views

            nxt = gb + 2

            @pl.when(nxt < GROUPS)
            def _refill(ib=ib, wb=wb, b=b, nxt=nxt):
                off = stream_base + nxt * CHUNK
                pltpu.async_copy(idx_hbm.at[pl.ds(off, CHUNK)], ib, isems[b])
                pltpu.async_copy(w_hbm.at[pl.ds(off, CHUNK)], wb, wsems[b])

            total = (accs[0] + accs[1]) + (accs[2] + accs[3])
            res = total * SCALE + bias_v[pl.ds(gb * L, L)]
            out_v[pl.ds(PIX_PER_W - L - gb * L, L)] = jnp.flip(res, 0)

    out_base = NPIX - base_p - PIX_PER_W
    pltpu.sync_copy(out_v, out_hbm.at[pl.ds(out_base, PIX_PER_W)])


def kernel(input, indices, weight, bias):
    flat = input.reshape(-1)
    bits = lax.bitcast_convert_type(flat, jnp.int32)
    lsb = lax.shift_right_logical(bits, 16) & 1
    r = bits + jnp.int32(0x7FFF) + lsb
    words = (lax.shift_right_logical(r[:TABLE_WORDS], 16)
             | (r[TABLE_WORDS:] & jnp.int32(-65536)))
    out = _backproject(words, indices, weight, bias)
    return out.reshape(1, 1, NX, NY)


# speedup vs baseline: 1462.8330x; 1.0432x over previous
import functools

import jax
import jax.numpy as jnp
import numpy as np
from jax import lax
from jax.experimental import pallas as pl
from jax.experimental.pallas import tpu as pltpu
from jax.experimental.pallas import tpu_sc as plsc

VIEWS = 360
NDETU = 512
NX = 256
NY = 256
NPIX = NX * NY
TABLE_WORDS = VIEWS * NDETU // 2
SCALE = float(2.0 * np.pi) / (2.0 * VIEWS)

NC = 2
NS = 16
L = 16
NW = NC * NS
PIX_PER_W = NPIX // NW
GROUPS = PIX_PER_W // L
CHUNK = L * VIEWS
VSTEP = 8

_MESH = plsc.VectorSubcoreMesh(
    core_axis_name="c", subcore_axis_name="s", num_cores=NC, num_subcores=NS)


@functools.partial(
    pl.kernel,
    out_type=jax.ShapeDtypeStruct((NPIX,), jnp.float32),
    mesh=_MESH,
    compiler_params=pltpu.CompilerParams(needs_layout_passes=False),
    scratch_types=[
        pltpu.VMEM((TABLE_WORDS,), jnp.int32),
        pltpu.VMEM((CHUNK,), jnp.int32),
        pltpu.VMEM((CHUNK,), jnp.int32),
        pltpu.VMEM((CHUNK,), jnp.float32),
        pltpu.VMEM((CHUNK,), jnp.float32),
        pltpu.VMEM((PIX_PER_W,), jnp.float32),
        pltpu.VMEM((PIX_PER_W,), jnp.float32),
        pltpu.SemaphoreType.DMA,
        pltpu.SemaphoreType.DMA,
        pltpu.SemaphoreType.DMA,
        pltpu.SemaphoreType.DMA,
    ],
)
def _backproject(words_hbm, idx_hbm, w_hbm, bias_hbm, out_hbm,
                 table_v, idx0, idx1, w0, w1, bias_v, out_v,
                 si0, si1, sw0, sw1):
    wid = lax.axis_index("s") * NC + lax.axis_index("c")
    base_p = wid * PIX_PER_W
    stream_base = base_p * VIEWS

    idx_bufs = (idx0, idx1)
    w_bufs = (w0, w1)
    isems = (si0, si1)
    wsems = (sw0, sw1)

    for b in range(2):
        off = stream_base + b * CHUNK
        pltpu.async_copy(idx_hbm.at[pl.ds(off, CHUNK)], idx_bufs[b], isems[b])
        pltpu.async_copy(w_hbm.at[pl.ds(off, CHUNK)], w_bufs[b], wsems[b])
    pltpu.sync_copy(words_hbm, table_v)
    pltpu.sync_copy(bias_hbm.at[pl.ds(base_p, PIX_PER_W)], bias_v)

    lane_base = lax.iota(jnp.int32, L) * VIEWS

    @pl.loop(0, GROUPS, step=2)
    def _group_pair(g):
        for b in range(2):
            gb = g + b
            ib, wb = idx_bufs[b], w_bufs[b]
            pltpu.make_async_copy(
                idx_hbm.at[pl.ds(0, CHUNK)], ib, isems[b]).wait()
            pltpu.make_async_copy(
                w_hbm.at[pl.ds(0, CHUNK)], wb, wsems[b]).wait()

            def _views(t, accs, ib=ib, wb=wb):
                lo_t = lane_base + t * VSTEP
                new = []
                for j in range(VSTEP):
                    lo = lo_t + j
                    iv = plsc.load_gather(ib, [lo])
                    wv = plsc.load_gather(wb, [lo])
                    hi = iv >= TABLE_WORDS
                    word = plsc.load_gather(
                        table_v, [jnp.where(hi, iv - TABLE_WORDS, iv)])
                    bits = jnp.where(hi, word & jnp.int32(-65536), word << 16)
                    x = plsc.bitcast(bits, jnp.float32)
                    new.append(accs[j] + wv * x)
                return tuple(new)

            z = jnp.zeros((L,), jnp.float32)
            accs = lax.fori_loop(0, VIEWS // VSTEP, _views, (z,) * VSTEP)

            nxt = gb + 2

            @pl.when(nxt < GROUPS)
            def _refill(ib=ib, wb=wb, b=b, nxt=nxt):
                off = stream_base + nxt * CHUNK
                pltpu.async_copy(idx_hbm.at[pl.ds(off, CHUNK)], ib, isems[b])
                pltpu.async_copy(w_hbm.at[pl.ds(off, CHUNK)], wb, wsems[b])

            pairs = list(accs)
            while len(pairs) > 1:
                pairs = [pairs[i] + pairs[i + 1]
                         for i in range(0, len(pairs), 2)]
            total = pairs[0]
            res = total * SCALE + bias_v[pl.ds(gb * L, L)]
            out_v[pl.ds(PIX_PER_W - L - gb * L, L)] = jnp.flip(res, 0)

    out_base = NPIX - base_p - PIX_PER_W
    pltpu.sync_copy(out_v, out_hbm.at[pl.ds(out_base, PIX_PER_W)])


def kernel(input, indices, weight, bias):
    flat = input.reshape(-1)
    bits = lax.bitcast_convert_type(flat, jnp.int32)
    lsb = lax.shift_right_logical(bits, 16) & 1
    r = bits + jnp.int32(0x7FFF) + lsb
    words = (lax.shift_right_logical(r[:TABLE_WORDS], 16)
             | (r[TABLE_WORDS:] & jnp.int32(-65536)))
    out = _backproject(words, indices, weight, bias)
    return out.reshape(1, 1, NX, NY)
